# Initial kernel scaffold; baseline (speedup 1.0000x reference)
import jax
import jax.numpy as jnp
from jax.experimental import pallas as pl


def kernel(x):
    raise NotImplementedError("write your pallas kernel here")


# speedup vs baseline: 3.3251x; 3.3251x over previous
import functools

import jax
import jax.numpy as jnp
from jax import lax
from jax.experimental import pallas as pl
from jax.experimental.pallas import tpu as pltpu
from jax.experimental.pallas import tpu_sc as plsc

_TOPK = 1024
_B = 128
_N = 32768
_LANES = 16
_NCORES = 2
_NSUB = 16
_NWORKERS = _NCORES * _NSUB
_ROWS_PER_W = _B // _NWORKERS
_NVREGS = _N // _LANES
_CBUF = _N + 32 * _LANES + _LANES

_MIN32 = -2147483648


def _key_of(v):
    b = plsc.bitcast(v, jnp.int32)
    return jnp.where(b < 0, ~b, b ^ jnp.int32(_MIN32))


def _bit_of(u, b):
    return jnp.bitwise_and(lax.shift_right_logical(u, jnp.int32(b)), jnp.int32(1))


def _topk_mask_kernel(x_hbm, out_hbm, row_v, cand_a, cand_b, *, nrows):
    wid = lax.axis_index("c") * _NSUB + lax.axis_index("s")
    zeros16 = jnp.zeros((_LANES,), jnp.int32)

    def do_row(r, carry):
        row = wid * nrows + r
        pltpu.sync_copy(x_hbm.at[row], row_v)

        def count_body(i, acc):
            u = _key_of(row_v[pl.ds(i * _LANES, _LANES)])
            return acc + _bit_of(u, 31)

        acc = lax.fori_loop(0, _NVREGS, count_body, zeros16)
        cnt_set = jnp.sum(acc)

        def run_pass(src, dst, n, bit, tbit, src_f32):
            ntrips = (n + _LANES - 1) // _LANES

            def body(i, st):
                wptr, acc = st
                if src_f32:
                    u = _key_of(src[pl.ds(i * _LANES, _LANES)])
                else:
                    u = src[pl.ds(i * _LANES, _LANES)]
                keep = _bit_of(u, bit) == tbit
                offs = plsc.cumsum(jnp.where(keep, jnp.int32(1), jnp.int32(0)))
                dest = wptr + offs - 1
                plsc.store_scatter(dst, [dest], u, mask=keep)
                nxt = jnp.where(keep, _bit_of(u, bit - 1), jnp.int32(0))
                pcnt = plsc.all_reduce_population_count(keep)
                return wptr + pcnt, acc + nxt

            wptr, acc = lax.fori_loop(0, ntrips, body, (zeros16, zeros16))
            n_new = jnp.max(wptr)
            pad_idx = wptr + lax.iota(jnp.int32, _LANES)
            plsc.store_scatter(dst, [pad_idx], zeros16)
            return n_new, jnp.sum(acc)

        need = jnp.int32(_TOPK)
        n = jnp.int32(_N)
        tbit = jnp.where(cnt_set >= need, jnp.int32(1), jnp.int32(0))
        prefix = lax.shift_left(tbit, jnp.int32(31))
        need = need - jnp.where(tbit == 0, cnt_set, jnp.int32(0))
        n, cnt_set = run_pass(row_v, cand_a, n, 31, tbit, True)

        src, dst = cand_a, cand_b
        for bit in range(30, 0, -1):
            tbit = jnp.where(cnt_set >= need, jnp.int32(1), jnp.int32(0))
            prefix = prefix | lax.shift_left(tbit, jnp.int32(bit))
            need = need - jnp.where(tbit == 0, cnt_set, jnp.int32(0))
            n, cnt_set = run_pass(src, dst, n, bit, tbit, False)
            src, dst = dst, src

        tbit = jnp.where(cnt_set >= need, jnp.int32(1), jnp.int32(0))
        prefix = prefix | tbit

        t_rank = prefix ^ jnp.int32(_MIN32)

        def out_body(i, c):
            sl = pl.ds(i * _LANES, _LANES)
            v = row_v[sl]
            rank = _key_of(v) ^ _MIN32
            row_v[sl] = jnp.where(rank >= t_rank, v, jnp.float32(0.0))
            return c

        lax.fori_loop(0, _NVREGS, out_body, jnp.int32(0))
        pltpu.sync_copy(row_v, out_hbm.at[row])
        return carry

    lax.fori_loop(0, nrows, do_row, jnp.int32(0))


@jax.jit
def kernel(x):
    mesh = plsc.VectorSubcoreMesh(core_axis_name="c", subcore_axis_name="s")
    k = pl.kernel(
        functools.partial(_topk_mask_kernel, nrows=_ROWS_PER_W),
        out_type=jax.ShapeDtypeStruct((_B, _N), jnp.float32),
        mesh=mesh,
        compiler_params=pltpu.CompilerParams(needs_layout_passes=False),
        scratch_types=[
            pltpu.VMEM((_N,), jnp.float32),
            pltpu.VMEM((_CBUF,), jnp.int32),
            pltpu.VMEM((_CBUF,), jnp.int32),
        ],
    )
    return k(x)
